# baseline (speedup 1.0000x reference)
import functools

import jax
import jax.numpy as jnp
from jax import lax
from jax.experimental import pallas as pl
from jax.experimental.pallas import tpu as pltpu
from jax.experimental.pallas import tpu_sc as plsc

N = 10000
D = 128
NC = 2
NS = 16
NW = NC * NS
NPAD = 10240
ROWS_PER_TILE = NPAD // NS
E = 320000
EPW = 10240
EPAD = NW * EPW
DCH = 128
DNCH = EPW // DCH
CH = 80
NCHUNK0 = 200
NCHUNK1 = 52
ECHUNKS = NS * (NCHUNK0 + NCHUNK1)
EPAD2 = ECHUNKS * CH
NB = 4

_mesh = plsc.VectorSubcoreMesh(
    core_axis_name="c", subcore_axis_name="s", num_cores=NC, num_subcores=NS
)


@functools.partial(
    pl.kernel,
    out_type=(
        jax.ShapeDtypeStruct((NPAD,), jnp.float32),
        jax.ShapeDtypeStruct((NPAD,), jnp.float32),
    ),
    mesh=_mesh,
    scratch_types=[
        pltpu.VMEM((NB, DCH), jnp.int32),
        pltpu.VMEM((DCH,), jnp.float32),
        pltpu.VMEM((DCH,), jnp.float32),
        pltpu.VMEM_SHARED((NPAD,), jnp.float32),
        [pltpu.SemaphoreType.DMA] * NB,
    ],
)
def _deg_sc(dst_hbm, deg0_hbm, deg1_hbm, idx_v, ones_v, zeros_v, deg_sh,
            dsems):
    c = lax.axis_index("c")
    s = lax.axis_index("s")
    wid = c * NS + s
    ebase = wid * EPW

    def idx_start(b, j):
        pltpu.async_copy(dst_hbm.at[pl.ds(ebase + j * DCH, DCH)],
                         idx_v.at[b], dsems[b])

    def idx_wait(b, j):
        pltpu.make_async_copy(dst_hbm.at[pl.ds(ebase + j * DCH, DCH)],
                              idx_v.at[b], dsems[b]).wait()

    for b in range(NB):
        idx_start(b, b)
    for k in range(DCH // 16):
        ones_v[pl.ds(k * 16, 16)] = jnp.ones((16,), jnp.float32)
        zeros_v[pl.ds(k * 16, 16)] = jnp.zeros((16,), jnp.float32)
    for k in range(ROWS_PER_TILE // DCH):
        pltpu.sync_copy(zeros_v, deg_sh.at[pl.ds(s * ROWS_PER_TILE + k * DCH, DCH)])
    plsc.subcore_barrier()

    def body(g, carry):
        for b in range(NB):
            j = g * NB + b
            idx_wait(b, j)
            pltpu.sync_copy(ones_v, deg_sh.at[idx_v.at[b]], add=True)
            idx_start(b, j + NB)
        return carry

    lax.fori_loop(0, DNCH // NB - 1, body, 0)
    for b in range(NB):
        j = DNCH - NB + b
        idx_wait(b, j)
        pltpu.sync_copy(ones_v, deg_sh.at[idx_v.at[b]], add=True)
    plsc.subcore_barrier()
    sl = pl.ds(s * ROWS_PER_TILE, ROWS_PER_TILE)

    @pl.when(c == 0)
    def _():
        pltpu.sync_copy(deg_sh.at[sl], deg0_hbm.at[sl])

    @pl.when(c == 1)
    def _():
        pltpu.sync_copy(deg_sh.at[sl], deg1_hbm.at[sl])


NG0 = NCHUNK0 // NB
NG1 = NCHUNK1 // NB


@functools.partial(
    pl.kernel,
    out_type=(
        jax.ShapeDtypeStruct((NPAD, D), jnp.float32),
        jax.ShapeDtypeStruct((NPAD, D), jnp.float32),
    ),
    mesh=_mesh,
    scratch_types=[
        pltpu.VMEM((NB, 2, CH), jnp.int32),
        pltpu.VMEM((NB, CH, D), jnp.float32),
        pltpu.VMEM_SHARED((NPAD, D), jnp.float32),
        [pltpu.SemaphoreType.DMA] * NB,
        [pltpu.SemaphoreType.DMA] * NB,
        [pltpu.SemaphoreType.DMA] * NB,
    ],
)
def _edge_sc(y_hbm, sd_hbm, acc0_hbm, acc1_hbm,
             sd_v, rows_v, acc_sh, isems, gsems, ssems):
    c = lax.axis_index("c")
    s = lax.axis_index("s")

    def _run(yref, cbase, ng):
        def idx_start(b, j):
            pltpu.async_copy(sd_hbm.at[cbase + j], sd_v.at[b], isems[b])

        def idx_wait(b, j):
            pltpu.make_async_copy(sd_hbm.at[cbase + j], sd_v.at[b],
                                  isems[b]).wait()

        def gather_start(b):
            pltpu.async_copy(yref.at[sd_v.at[b, 0]], rows_v.at[b], gsems[b])

        def gather_wait(b):
            pltpu.make_async_copy(yref.at[sd_v.at[b, 0]], rows_v.at[b],
                                  gsems[b]).wait()

        for b in range(NB):
            idx_start(b, b)
        for b in range(NB):
            idx_wait(b, b)
            gather_start(b)

        def scatter_start(b):
            return pltpu.async_copy(rows_v.at[b], acc_sh.at[sd_v.at[b, 1]],
                                    ssems[b], add=True)

        def body(g, carry):
            descs = []
            for b in range(NB):
                gather_wait(b)
                if b > 0:
                    descs[b - 1].wait()
                    idx_start(b - 1, (g + 1) * NB + b - 1)
                descs.append(scatter_start(b))
            descs[NB - 1].wait()
            idx_start(NB - 1, (g + 1) * NB + NB - 1)
            for b in range(NB):
                idx_wait(b, (g + 1) * NB + b)
                gather_start(b)
            return carry

        lax.fori_loop(0, ng - 1, body, 0)
        descs = []
        for b in range(NB):
            gather_wait(b)
            if b > 0:
                descs[b - 1].wait()
            descs.append(scatter_start(b))
        descs[NB - 1].wait()

    def zbody(r, carry):
        for k in range(D // 16):
            rows_v[0, r, pl.ds(k * 16, 16)] = jnp.zeros((16,), jnp.float32)
        return carry

    lax.fori_loop(0, CH, zbody, 0)
    for k in range(ROWS_PER_TILE // CH):
        pltpu.sync_copy(rows_v.at[0],
                        acc_sh.at[pl.ds(s * ROWS_PER_TILE + k * CH, CH)])
    plsc.subcore_barrier()

    @pl.when(c == 0)
    def _():
        _run(y_hbm, s * NCHUNK0, NG0)

    @pl.when(c == 1)
    def _():
        _run(y_hbm, NS * NCHUNK0 + s * NCHUNK1, NG1)

    plsc.subcore_barrier()
    sl = pl.ds(s * ROWS_PER_TILE, ROWS_PER_TILE)

    @pl.when(c == 0)
    def _():
        pltpu.sync_copy(acc_sh.at[sl], acc0_hbm.at[sl])

    @pl.when(c == 1)
    def _():
        pltpu.sync_copy(acc_sh.at[sl], acc1_hbm.at[sl])


BS = 200


def _matmul_body(x_ref, w_ref, xw_ref):
    xw_ref[...] = jnp.dot(x_ref[...], w_ref[...],
                          preferred_element_type=jnp.float32)


def _matmul_tc(x_p, W_conv):
    return pl.pallas_call(
        _matmul_body,
        grid=(N // BS,),
        in_specs=[
            pl.BlockSpec((BS, D), lambda i: (i, 0)),
            pl.BlockSpec((D, D), lambda i: (0, 0)),
        ],
        out_specs=pl.BlockSpec((BS, D), lambda i: (i, 0)),
        out_shape=jax.ShapeDtypeStruct((N, D), jnp.float32),
    )(x_p, W_conv)


def _scale_body(xw_ref, d0_ref, d1_ref, y_ref):
    deg = d0_ref[...] + d1_ref[...] + 1.0
    dinv = lax.rsqrt(deg)
    y_ref[...] = xw_ref[...] * dinv


def _scale_tc(xw, deg0, deg1):
    grid = (N // BS,)
    return pl.pallas_call(
        _scale_body,
        grid=grid,
        in_specs=[
            pl.BlockSpec((BS, D), lambda i: (i, 0)),
            pl.BlockSpec((BS, 1), lambda i: (i, 0)),
            pl.BlockSpec((BS, 1), lambda i: (i, 0)),
        ],
        out_specs=pl.BlockSpec((BS, D), lambda i: (i, 0)),
        out_shape=jax.ShapeDtypeStruct((N, D), jnp.float32),
    )(xw, deg0, deg1)


def _combine_body(a0_ref, a1_ref, y_ref, d0_ref, d1_ref, bc_ref, wl_ref,
                  bl_ref, o_ref):
    deg = d0_ref[...] + d1_ref[...] + 1.0
    dinv = lax.rsqrt(deg)
    pre = (a0_ref[...] + a1_ref[...] + y_ref[...]) * dinv
    h = jnp.maximum(pre + bc_ref[...], 0.0)
    o_ref[...] = (
        jnp.dot(h, wl_ref[...], preferred_element_type=jnp.float32)
        + bl_ref[...]
    )


def _combine_tc(acc0, acc1, y, deg0, deg1, b_conv, W_lin, b_lin):
    grid = (N // BS,)
    blk = pl.BlockSpec((BS, D), lambda i: (i, 0))
    return pl.pallas_call(
        _combine_body,
        grid=grid,
        in_specs=[
            blk, blk, blk,
            pl.BlockSpec((BS, 1), lambda i: (i, 0)),
            pl.BlockSpec((BS, 1), lambda i: (i, 0)),
            pl.BlockSpec((1, D), lambda i: (0, 0)),
            pl.BlockSpec((D, D), lambda i: (0, 0)),
            pl.BlockSpec((1, D), lambda i: (0, 0)),
        ],
        out_specs=blk,
        out_shape=jax.ShapeDtypeStruct((N, D), jnp.float32),
    )(acc0, acc1, y, deg0, deg1, b_conv, W_lin, b_lin)


def kernel(x, edge_index, W_conv, b_conv, W_lin, b_lin):
    src = edge_index[0].astype(jnp.int32)
    dst = edge_index[1].astype(jnp.int32)
    dst_p = jnp.concatenate([dst, jnp.full((EPAD - E,), N, jnp.int32)])
    sd_3d = jnp.stack(
        [jnp.concatenate([src, jnp.zeros((EPAD2 - E,), jnp.int32)]
                         ).reshape(ECHUNKS, CH),
         jnp.concatenate([dst, jnp.full((EPAD2 - E,), N, jnp.int32)]
                         ).reshape(ECHUNKS, CH)],
        axis=1)

    xw = _matmul_tc(x, W_conv)
    deg0, deg1 = _deg_sc(dst_p)
    deg0 = deg0.reshape(NPAD, 1)
    deg1 = deg1.reshape(NPAD, 1)
    y = _scale_tc(xw, deg0, deg1)
    acc0, acc1 = _edge_sc(y, sd_3d)
    out = _combine_tc(acc0, acc1, y, deg0, deg1,
                      b_conv.reshape(1, D), W_lin, b_lin.reshape(1, D))
    return out


# speedup vs baseline: 1.1256x; 1.1256x over previous
import functools

import jax
import jax.numpy as jnp
from jax import lax
from jax.experimental import pallas as pl
from jax.experimental.pallas import tpu as pltpu
from jax.experimental.pallas import tpu_sc as plsc

N = 10000
D = 128
NC = 2
NS = 16
NW = NC * NS
NPAD = 10240
ROWS_PER_TILE = NPAD // NS
E = 320000
EPW = 10240
EPAD = NW * EPW
DCH = 128
DNCH = EPW // DCH
CH = 80
NCHUNK0 = 200
NCHUNK1 = 52
ECHUNKS = NS * (NCHUNK0 + NCHUNK1)
EPAD2 = ECHUNKS * CH
NB = 4

_mesh = plsc.VectorSubcoreMesh(
    core_axis_name="c", subcore_axis_name="s", num_cores=NC, num_subcores=NS
)


@functools.partial(
    pl.kernel,
    out_type=(
        jax.ShapeDtypeStruct((NPAD,), jnp.float32),
        jax.ShapeDtypeStruct((NPAD,), jnp.float32),
    ),
    mesh=_mesh,
    scratch_types=[
        pltpu.VMEM((NB, DCH), jnp.int32),
        pltpu.VMEM((DCH,), jnp.float32),
        pltpu.VMEM((DCH,), jnp.float32),
        pltpu.VMEM_SHARED((NPAD,), jnp.float32),
        [pltpu.SemaphoreType.DMA] * NB,
    ],
)
def _deg_sc(dst_hbm, deg0_hbm, deg1_hbm, idx_v, ones_v, zeros_v, deg_sh,
            dsems):
    c = lax.axis_index("c")
    s = lax.axis_index("s")
    wid = c * NS + s
    ebase = wid * EPW

    def idx_start(b, j):
        pltpu.async_copy(dst_hbm.at[pl.ds(ebase + j * DCH, DCH)],
                         idx_v.at[b], dsems[b])

    def idx_wait(b, j):
        pltpu.make_async_copy(dst_hbm.at[pl.ds(ebase + j * DCH, DCH)],
                              idx_v.at[b], dsems[b]).wait()

    for b in range(NB):
        idx_start(b, b)
    for k in range(DCH // 16):
        ones_v[pl.ds(k * 16, 16)] = jnp.ones((16,), jnp.float32)
        zeros_v[pl.ds(k * 16, 16)] = jnp.zeros((16,), jnp.float32)
    for k in range(ROWS_PER_TILE // DCH):
        pltpu.sync_copy(zeros_v, deg_sh.at[pl.ds(s * ROWS_PER_TILE + k * DCH, DCH)])
    plsc.subcore_barrier()

    def body(g, carry):
        for b in range(NB):
            j = g * NB + b
            idx_wait(b, j)
            pltpu.sync_copy(ones_v, deg_sh.at[idx_v.at[b]], add=True)
            idx_start(b, j + NB)
        return carry

    lax.fori_loop(0, DNCH // NB - 1, body, 0)
    for b in range(NB):
        j = DNCH - NB + b
        idx_wait(b, j)
        pltpu.sync_copy(ones_v, deg_sh.at[idx_v.at[b]], add=True)
    plsc.subcore_barrier()
    sl = pl.ds(s * ROWS_PER_TILE, ROWS_PER_TILE)

    @pl.when(c == 0)
    def _():
        pltpu.sync_copy(deg_sh.at[sl], deg0_hbm.at[sl])

    @pl.when(c == 1)
    def _():
        pltpu.sync_copy(deg_sh.at[sl], deg1_hbm.at[sl])


NG0 = NCHUNK0 // NB
NG1 = NCHUNK1 // NB


@functools.partial(
    pl.kernel,
    out_type=(
        jax.ShapeDtypeStruct((NPAD, D), jnp.float32),
        jax.ShapeDtypeStruct((NPAD, D), jnp.float32),
    ),
    mesh=_mesh,
    scratch_types=[
        pltpu.VMEM((NB, 2, CH), jnp.int32),
        pltpu.VMEM((NB, CH, D), jnp.float32),
        pltpu.VMEM_SHARED((NPAD, D), jnp.float32),
        [pltpu.SemaphoreType.DMA] * NB,
        [pltpu.SemaphoreType.DMA] * NB,
        [pltpu.SemaphoreType.DMA] * NB,
    ],
)
def _edge_sc(y_hbm, sd_hbm, acc0_hbm, acc1_hbm,
             sd_v, rows_v, acc_sh, isems, gsems, ssems):
    c = lax.axis_index("c")
    s = lax.axis_index("s")

    def _run(yref, cbase, ng):
        def idx_start(b, j):
            pltpu.async_copy(sd_hbm.at[cbase + j], sd_v.at[b], isems[b])

        def idx_wait(b, j):
            pltpu.make_async_copy(sd_hbm.at[cbase + j], sd_v.at[b],
                                  isems[b]).wait()

        def gather_start(b):
            pltpu.async_copy(yref.at[sd_v.at[b, 0]], rows_v.at[b], gsems[b])

        def gather_wait(b):
            pltpu.make_async_copy(yref.at[sd_v.at[b, 0]], rows_v.at[b],
                                  gsems[b]).wait()

        for b in range(NB):
            idx_start(b, b)
        for b in range(NB):
            idx_wait(b, b)
            gather_start(b)

        def scatter_start(b):
            return pltpu.async_copy(rows_v.at[b], acc_sh.at[sd_v.at[b, 1]],
                                    ssems[b], add=True)

        def body(g, carry):
            descs = []
            for b in range(NB):
                gather_wait(b)
                if b > 0:
                    descs[b - 1].wait()
                    idx_start(b - 1, (g + 1) * NB + b - 1)
                descs.append(scatter_start(b))
            descs[NB - 1].wait()
            idx_start(NB - 1, (g + 1) * NB + NB - 1)
            for b in range(NB):
                idx_wait(b, (g + 1) * NB + b)
                gather_start(b)
            return carry

        lax.fori_loop(0, ng - 1, body, 0)
        descs = []
        for b in range(NB):
            gather_wait(b)
            if b > 0:
                descs[b - 1].wait()
            descs.append(scatter_start(b))
        descs[NB - 1].wait()

    def zbody(r, carry):
        for k in range(D // 16):
            rows_v[0, r, pl.ds(k * 16, 16)] = jnp.zeros((16,), jnp.float32)
        return carry

    lax.fori_loop(0, CH, zbody, 0)
    for k in range(ROWS_PER_TILE // CH):
        pltpu.sync_copy(rows_v.at[0],
                        acc_sh.at[pl.ds(s * ROWS_PER_TILE + k * CH, CH)])
    plsc.subcore_barrier()

    @pl.when(c == 0)
    def _():
        _run(y_hbm, s * NCHUNK0, NG0)

    @pl.when(c == 1)
    def _():
        _run(y_hbm, NS * NCHUNK0 + s * NCHUNK1, NG1)

    plsc.subcore_barrier()
    sl = pl.ds(s * ROWS_PER_TILE, ROWS_PER_TILE)

    @pl.when(c == 0)
    def _():
        pltpu.sync_copy(acc_sh.at[sl], acc0_hbm.at[sl])

    @pl.when(c == 1)
    def _():
        pltpu.sync_copy(acc_sh.at[sl], acc1_hbm.at[sl])


BS = 400


def _matmul_body(x_ref, w_ref, xw_ref):
    xw_ref[...] = jnp.dot(x_ref[...], w_ref[...],
                          preferred_element_type=jnp.float32)


def _matmul_tc(x_p, W_conv):
    return pl.pallas_call(
        _matmul_body,
        grid=(N // BS,),
        in_specs=[
            pl.BlockSpec((BS, D), lambda i: (i, 0)),
            pl.BlockSpec((D, D), lambda i: (0, 0)),
        ],
        out_specs=pl.BlockSpec((BS, D), lambda i: (i, 0)),
        out_shape=jax.ShapeDtypeStruct((N, D), jnp.float32),
    )(x_p, W_conv)


def _scale_body(xw_ref, d0_ref, d1_ref, y_ref):
    deg = d0_ref[...] + d1_ref[...] + 1.0
    dinv = lax.rsqrt(deg)
    y_ref[...] = xw_ref[...] * dinv


def _scale_tc(xw, deg0, deg1):
    grid = (N // BS,)
    return pl.pallas_call(
        _scale_body,
        grid=grid,
        in_specs=[
            pl.BlockSpec((BS, D), lambda i: (i, 0)),
            pl.BlockSpec((BS, 1), lambda i: (i, 0)),
            pl.BlockSpec((BS, 1), lambda i: (i, 0)),
        ],
        out_specs=pl.BlockSpec((BS, D), lambda i: (i, 0)),
        out_shape=jax.ShapeDtypeStruct((N, D), jnp.float32),
    )(xw, deg0, deg1)


def _combine_body(a0_ref, a1_ref, y_ref, d0_ref, d1_ref, bc_ref, wl_ref,
                  bl_ref, o_ref):
    deg = d0_ref[...] + d1_ref[...] + 1.0
    dinv = lax.rsqrt(deg)
    pre = (a0_ref[...] + a1_ref[...] + y_ref[...]) * dinv
    h = jnp.maximum(pre + bc_ref[...], 0.0)
    o_ref[...] = (
        jnp.dot(h, wl_ref[...], preferred_element_type=jnp.float32)
        + bl_ref[...]
    )


def _combine_tc(acc0, acc1, y, deg0, deg1, b_conv, W_lin, b_lin):
    grid = (N // BS,)
    blk = pl.BlockSpec((BS, D), lambda i: (i, 0))
    return pl.pallas_call(
        _combine_body,
        grid=grid,
        in_specs=[
            blk, blk, blk,
            pl.BlockSpec((BS, 1), lambda i: (i, 0)),
            pl.BlockSpec((BS, 1), lambda i: (i, 0)),
            pl.BlockSpec((1, D), lambda i: (0, 0)),
            pl.BlockSpec((D, D), lambda i: (0, 0)),
            pl.BlockSpec((1, D), lambda i: (0, 0)),
        ],
        out_specs=blk,
        out_shape=jax.ShapeDtypeStruct((N, D), jnp.float32),
    )(acc0, acc1, y, deg0, deg1, b_conv, W_lin, b_lin)


def kernel(x, edge_index, W_conv, b_conv, W_lin, b_lin):
    src = edge_index[0].astype(jnp.int32)
    dst = edge_index[1].astype(jnp.int32)
    dst_p = jnp.concatenate([dst, jnp.full((EPAD - E,), N, jnp.int32)])
    sd_3d = jnp.stack(
        [jnp.concatenate([src, jnp.zeros((EPAD2 - E,), jnp.int32)]
                         ).reshape(ECHUNKS, CH),
         jnp.concatenate([dst, jnp.full((EPAD2 - E,), N, jnp.int32)]
                         ).reshape(ECHUNKS, CH)],
        axis=1)

    xw = _matmul_tc(x, W_conv)
    deg0, deg1 = _deg_sc(dst_p)
    deg0 = deg0.reshape(NPAD, 1)
    deg1 = deg1.reshape(NPAD, 1)
    y = _scale_tc(xw, deg0, deg1)
    acc0, acc1 = _edge_sc(y, sd_3d)
    out = _combine_tc(acc0, acc1, y, deg0, deg1,
                      b_conv.reshape(1, D), W_lin, b_lin.reshape(1, D))
    return out


# speedup vs baseline: 1.2159x; 1.0802x over previous
import functools

import jax
import jax.numpy as jnp
from jax import lax
from jax.experimental import pallas as pl
from jax.experimental.pallas import tpu as pltpu
from jax.experimental.pallas import tpu_sc as plsc

N = 10000
D = 128
NC = 2
NS = 16
NW = NC * NS
NPAD = 10240
ROWS_PER_TILE = NPAD // NS
E = 320000
EPW = 10240
EPAD = NW * EPW
DCH = 128
DNCH = EPW // DCH
CH = 80
NCHUNK0 = 200
NCHUNK1 = 52
ECHUNKS = NS * (NCHUNK0 + NCHUNK1)
EPAD2 = ECHUNKS * CH
NB = 4

_mesh = plsc.VectorSubcoreMesh(
    core_axis_name="c", subcore_axis_name="s", num_cores=NC, num_subcores=NS
)


@functools.partial(
    pl.kernel,
    out_type=(
        jax.ShapeDtypeStruct((NPAD,), jnp.float32),
        jax.ShapeDtypeStruct((NPAD,), jnp.float32),
    ),
    mesh=_mesh,
    scratch_types=[
        pltpu.VMEM((NB, DCH), jnp.int32),
        pltpu.VMEM((DCH,), jnp.float32),
        pltpu.VMEM((DCH,), jnp.float32),
        pltpu.VMEM_SHARED((NPAD,), jnp.float32),
        [pltpu.SemaphoreType.DMA] * NB,
    ],
)
def _deg_sc(dst_hbm, deg0_hbm, deg1_hbm, idx_v, ones_v, zeros_v, deg_sh,
            dsems):
    c = lax.axis_index("c")
    s = lax.axis_index("s")
    wid = c * NS + s
    ebase = wid * EPW

    def idx_start(b, j):
        pltpu.async_copy(dst_hbm.at[pl.ds(ebase + j * DCH, DCH)],
                         idx_v.at[b], dsems[b])

    def idx_wait(b, j):
        pltpu.make_async_copy(dst_hbm.at[pl.ds(ebase + j * DCH, DCH)],
                              idx_v.at[b], dsems[b]).wait()

    for b in range(NB):
        idx_start(b, b)
    for k in range(DCH // 16):
        ones_v[pl.ds(k * 16, 16)] = jnp.ones((16,), jnp.float32)
        zeros_v[pl.ds(k * 16, 16)] = jnp.zeros((16,), jnp.float32)
    for k in range(ROWS_PER_TILE // DCH):
        pltpu.sync_copy(zeros_v, deg_sh.at[pl.ds(s * ROWS_PER_TILE + k * DCH, DCH)])
    plsc.subcore_barrier()

    def body(g, carry):
        for b in range(NB):
            j = g * NB + b
            idx_wait(b, j)
            pltpu.sync_copy(ones_v, deg_sh.at[idx_v.at[b]], add=True)
            idx_start(b, j + NB)
        return carry

    lax.fori_loop(0, DNCH // NB - 1, body, 0)
    for b in range(NB):
        j = DNCH - NB + b
        idx_wait(b, j)
        pltpu.sync_copy(ones_v, deg_sh.at[idx_v.at[b]], add=True)
    plsc.subcore_barrier()
    sl = pl.ds(s * ROWS_PER_TILE, ROWS_PER_TILE)

    @pl.when(c == 0)
    def _():
        pltpu.sync_copy(deg_sh.at[sl], deg0_hbm.at[sl])

    @pl.when(c == 1)
    def _():
        pltpu.sync_copy(deg_sh.at[sl], deg1_hbm.at[sl])


NG0 = NCHUNK0 // NB
NG1 = NCHUNK1 // NB


@functools.partial(
    pl.kernel,
    out_type=(
        jax.ShapeDtypeStruct((NPAD, D), jnp.float32),
        jax.ShapeDtypeStruct((NPAD, D), jnp.float32),
    ),
    mesh=_mesh,
    scratch_types=[
        pltpu.VMEM((NB, 2, CH), jnp.int32),
        pltpu.VMEM((NB, CH, D), jnp.float32),
        pltpu.VMEM_SHARED((NPAD, D), jnp.float32),
        [pltpu.SemaphoreType.DMA] * NB,
        [pltpu.SemaphoreType.DMA] * NB,
        [pltpu.SemaphoreType.DMA] * NB,
    ],
)
def _edge_sc(y_hbm, sd_hbm, acc0_hbm, acc1_hbm,
             sd_v, rows_v, acc_sh, isems, gsems, ssems):
    c = lax.axis_index("c")
    s = lax.axis_index("s")

    def _run(yref, cbase, ng):
        def idx_start(b, j):
            pltpu.async_copy(sd_hbm.at[cbase + j], sd_v.at[b], isems[b])

        def idx_wait(b, j):
            pltpu.make_async_copy(sd_hbm.at[cbase + j], sd_v.at[b],
                                  isems[b]).wait()

        def gather_start(b):
            pltpu.async_copy(yref.at[sd_v.at[b, 0]], rows_v.at[b], gsems[b])

        def gather_wait(b):
            pltpu.make_async_copy(yref.at[sd_v.at[b, 0]], rows_v.at[b],
                                  gsems[b]).wait()

        for b in range(NB):
            idx_start(b, b)
        for b in range(NB):
            idx_wait(b, b)
            gather_start(b)

        def scatter_start(b):
            return pltpu.async_copy(rows_v.at[b], acc_sh.at[sd_v.at[b, 1]],
                                    ssems[b], add=True)

        def body(g, carry):
            descs = []
            for b in range(NB):
                gather_wait(b)
                if b > 0:
                    descs[b - 1].wait()
                    idx_start(b - 1, (g + 1) * NB + b - 1)
                descs.append(scatter_start(b))
            descs[NB - 1].wait()
            idx_start(NB - 1, (g + 1) * NB + NB - 1)
            for b in range(NB):
                idx_wait(b, (g + 1) * NB + b)
                gather_start(b)
            return carry

        lax.fori_loop(0, ng - 1, body, 0)
        descs = []
        for b in range(NB):
            gather_wait(b)
            if b > 0:
                descs[b - 1].wait()
            descs.append(scatter_start(b))
        descs[NB - 1].wait()

    def zbody(r, carry):
        for k in range(D // 16):
            rows_v[0, r, pl.ds(k * 16, 16)] = jnp.zeros((16,), jnp.float32)
        return carry

    lax.fori_loop(0, CH, zbody, 0)
    for k in range(ROWS_PER_TILE // CH):
        pltpu.sync_copy(rows_v.at[0],
                        acc_sh.at[pl.ds(s * ROWS_PER_TILE + k * CH, CH)])
    plsc.subcore_barrier()

    @pl.when(c == 0)
    def _():
        _run(y_hbm, s * NCHUNK0, NG0)

    @pl.when(c == 1)
    def _():
        _run(y_hbm, NS * NCHUNK0 + s * NCHUNK1, NG1)

    plsc.subcore_barrier()
    sl = pl.ds(s * ROWS_PER_TILE, ROWS_PER_TILE)

    @pl.when(c == 0)
    def _():
        pltpu.sync_copy(acc_sh.at[sl], acc0_hbm.at[sl])

    @pl.when(c == 1)
    def _():
        pltpu.sync_copy(acc_sh.at[sl], acc1_hbm.at[sl])


BS = 1000


def _matmul_body(x_ref, w_ref, xw_ref):
    xw_ref[...] = jnp.dot(x_ref[...], w_ref[...],
                          preferred_element_type=jnp.float32)


def _matmul_tc(x_p, W_conv):
    return pl.pallas_call(
        _matmul_body,
        grid=(N // BS,),
        in_specs=[
            pl.BlockSpec((BS, D), lambda i: (i, 0)),
            pl.BlockSpec((D, D), lambda i: (0, 0)),
        ],
        out_specs=pl.BlockSpec((BS, D), lambda i: (i, 0)),
        out_shape=jax.ShapeDtypeStruct((N, D), jnp.float32),
    )(x_p, W_conv)


def _scale_body(xw_ref, d0_ref, d1_ref, y_ref):
    deg = d0_ref[...] + d1_ref[...] + 1.0
    dinv = lax.rsqrt(deg)
    y_ref[...] = xw_ref[...] * dinv


def _scale_tc(xw, deg0, deg1):
    grid = (N // BS,)
    return pl.pallas_call(
        _scale_body,
        grid=grid,
        in_specs=[
            pl.BlockSpec((BS, D), lambda i: (i, 0)),
            pl.BlockSpec((BS, 1), lambda i: (i, 0)),
            pl.BlockSpec((BS, 1), lambda i: (i, 0)),
        ],
        out_specs=pl.BlockSpec((BS, D), lambda i: (i, 0)),
        out_shape=jax.ShapeDtypeStruct((N, D), jnp.float32),
    )(xw, deg0, deg1)


def _combine_body(a0_ref, a1_ref, y_ref, d0_ref, d1_ref, bc_ref, wl_ref,
                  bl_ref, o_ref):
    deg = d0_ref[...] + d1_ref[...] + 1.0
    dinv = lax.rsqrt(deg)
    pre = (a0_ref[...] + a1_ref[...] + y_ref[...]) * dinv
    h = jnp.maximum(pre + bc_ref[...], 0.0)
    o_ref[...] = (
        jnp.dot(h, wl_ref[...], preferred_element_type=jnp.float32)
        + bl_ref[...]
    )


def _combine_tc(acc0, acc1, y, deg0, deg1, b_conv, W_lin, b_lin):
    grid = (N // BS,)
    blk = pl.BlockSpec((BS, D), lambda i: (i, 0))
    return pl.pallas_call(
        _combine_body,
        grid=grid,
        in_specs=[
            blk, blk, blk,
            pl.BlockSpec((BS, 1), lambda i: (i, 0)),
            pl.BlockSpec((BS, 1), lambda i: (i, 0)),
            pl.BlockSpec((1, D), lambda i: (0, 0)),
            pl.BlockSpec((D, D), lambda i: (0, 0)),
            pl.BlockSpec((1, D), lambda i: (0, 0)),
        ],
        out_specs=blk,
        out_shape=jax.ShapeDtypeStruct((N, D), jnp.float32),
    )(acc0, acc1, y, deg0, deg1, b_conv, W_lin, b_lin)


def kernel(x, edge_index, W_conv, b_conv, W_lin, b_lin):
    src = edge_index[0].astype(jnp.int32)
    dst = edge_index[1].astype(jnp.int32)
    dst_p = jnp.concatenate([dst, jnp.full((EPAD - E,), N, jnp.int32)])
    sd_3d = jnp.stack(
        [jnp.concatenate([src, jnp.zeros((EPAD2 - E,), jnp.int32)]
                         ).reshape(ECHUNKS, CH),
         jnp.concatenate([dst, jnp.full((EPAD2 - E,), N, jnp.int32)]
                         ).reshape(ECHUNKS, CH)],
        axis=1)

    xw = _matmul_tc(x, W_conv)
    deg0, deg1 = _deg_sc(dst_p)
    deg0 = deg0.reshape(NPAD, 1)
    deg1 = deg1.reshape(NPAD, 1)
    y = _scale_tc(xw, deg0, deg1)
    acc0, acc1 = _edge_sc(y, sd_3d)
    out = _combine_tc(acc0, acc1, y, deg0, deg1,
                      b_conv.reshape(1, D), W_lin, b_lin.reshape(1, D))
    return out


# speedup vs baseline: 1.2380x; 1.0182x over previous
import functools

import jax
import jax.numpy as jnp
from jax import lax
from jax.experimental import pallas as pl
from jax.experimental.pallas import tpu as pltpu
from jax.experimental.pallas import tpu_sc as plsc

N = 10000
D = 128
NC = 2
NS = 16
NW = NC * NS
NPAD = 10240
ROWS_PER_TILE = NPAD // NS
E = 320000
EPW = 10240
EPAD = NW * EPW
DCH = 128
DNCH = EPW // DCH
CH = 80
NCHUNK0 = 200
NCHUNK1 = 52
ECHUNKS = NS * (NCHUNK0 + NCHUNK1)
EPAD2 = ECHUNKS * CH
NB = 4

_mesh = plsc.VectorSubcoreMesh(
    core_axis_name="c", subcore_axis_name="s", num_cores=NC, num_subcores=NS
)


@functools.partial(
    pl.kernel,
    out_type=(
        jax.ShapeDtypeStruct((NPAD,), jnp.float32),
        jax.ShapeDtypeStruct((NPAD,), jnp.float32),
    ),
    mesh=_mesh,
    scratch_types=[
        pltpu.VMEM((NB, DCH), jnp.int32),
        pltpu.VMEM((DCH,), jnp.float32),
        pltpu.VMEM((DCH,), jnp.float32),
        pltpu.VMEM_SHARED((NPAD,), jnp.float32),
        [pltpu.SemaphoreType.DMA] * NB,
    ],
)
def _deg_sc(dst_hbm, deg0_hbm, deg1_hbm, idx_v, ones_v, zeros_v, deg_sh,
            dsems):
    c = lax.axis_index("c")
    s = lax.axis_index("s")
    wid = c * NS + s
    ebase = wid * EPW

    def idx_start(b, j):
        pltpu.async_copy(dst_hbm.at[pl.ds(ebase + j * DCH, DCH)],
                         idx_v.at[b], dsems[b])

    def idx_wait(b, j):
        pltpu.make_async_copy(dst_hbm.at[pl.ds(ebase + j * DCH, DCH)],
                              idx_v.at[b], dsems[b]).wait()

    for b in range(NB):
        idx_start(b, b)
    for k in range(DCH // 16):
        ones_v[pl.ds(k * 16, 16)] = jnp.ones((16,), jnp.float32)
        zeros_v[pl.ds(k * 16, 16)] = jnp.zeros((16,), jnp.float32)
    for k in range(ROWS_PER_TILE // DCH):
        pltpu.sync_copy(zeros_v, deg_sh.at[pl.ds(s * ROWS_PER_TILE + k * DCH, DCH)])
    plsc.subcore_barrier()

    def body(g, carry):
        for b in range(NB):
            j = g * NB + b
            idx_wait(b, j)
            pltpu.sync_copy(ones_v, deg_sh.at[idx_v.at[b]], add=True)
            idx_start(b, j + NB)
        return carry

    lax.fori_loop(0, DNCH // NB - 1, body, 0)
    for b in range(NB):
        j = DNCH - NB + b
        idx_wait(b, j)
        pltpu.sync_copy(ones_v, deg_sh.at[idx_v.at[b]], add=True)
    plsc.subcore_barrier()
    sl = pl.ds(s * ROWS_PER_TILE, ROWS_PER_TILE)

    @pl.when(c == 0)
    def _():
        pltpu.sync_copy(deg_sh.at[sl], deg0_hbm.at[sl])

    @pl.when(c == 1)
    def _():
        pltpu.sync_copy(deg_sh.at[sl], deg1_hbm.at[sl])


NG0 = NCHUNK0 // NB
NG1 = NCHUNK1 // NB


@functools.partial(
    pl.kernel,
    out_type=(
        jax.ShapeDtypeStruct((NPAD, D), jnp.float32),
        jax.ShapeDtypeStruct((NPAD, D), jnp.float32),
    ),
    mesh=_mesh,
    scratch_types=[
        pltpu.VMEM((NB, 2, CH), jnp.int32),
        pltpu.VMEM((NB, CH, D), jnp.float32),
        pltpu.VMEM_SHARED((NPAD, D), jnp.float32),
        [pltpu.SemaphoreType.DMA] * NB,
        [pltpu.SemaphoreType.DMA] * NB,
        [pltpu.SemaphoreType.DMA] * NB,
    ],
)
def _edge_sc(y_hbm, sd_hbm, acc0_hbm, acc1_hbm,
             sd_v, rows_v, acc_sh, isems, gsems, ssems):
    c = lax.axis_index("c")
    s = lax.axis_index("s")

    def _run(yref, cbase, ng):
        def idx_start(b, j):
            pltpu.async_copy(sd_hbm.at[cbase + j], sd_v.at[b], isems[b])

        def idx_wait(b, j):
            pltpu.make_async_copy(sd_hbm.at[cbase + j], sd_v.at[b],
                                  isems[b]).wait()

        def gather_start(b):
            pltpu.async_copy(yref.at[sd_v.at[b, 0]], rows_v.at[b], gsems[b])

        def gather_wait(b):
            pltpu.make_async_copy(yref.at[sd_v.at[b, 0]], rows_v.at[b],
                                  gsems[b]).wait()

        for b in range(NB):
            idx_start(b, b)
        for b in range(NB):
            idx_wait(b, b)
            gather_start(b)

        def scatter_start(b):
            return pltpu.async_copy(rows_v.at[b], acc_sh.at[sd_v.at[b, 1]],
                                    ssems[b], add=True)

        def body(g, carry):
            descs = []
            for b in range(NB):
                gather_wait(b)
                if b > 0:
                    descs[b - 1].wait()
                    idx_start(b - 1, (g + 1) * NB + b - 1)
                descs.append(scatter_start(b))
            descs[NB - 1].wait()
            idx_start(NB - 1, (g + 1) * NB + NB - 1)
            for b in range(NB):
                idx_wait(b, (g + 1) * NB + b)
                gather_start(b)
            return carry

        lax.fori_loop(0, ng - 1, body, 0)
        descs = []
        for b in range(NB):
            gather_wait(b)
            if b > 0:
                descs[b - 1].wait()
            descs.append(scatter_start(b))
        descs[NB - 1].wait()

    def zbody(r, carry):
        for k in range(D // 16):
            rows_v[0, r, pl.ds(k * 16, 16)] = jnp.zeros((16,), jnp.float32)
        return carry

    lax.fori_loop(0, CH, zbody, 0)
    for k in range(ROWS_PER_TILE // CH):
        pltpu.sync_copy(rows_v.at[0],
                        acc_sh.at[pl.ds(s * ROWS_PER_TILE + k * CH, CH)])
    plsc.subcore_barrier()

    @pl.when(c == 0)
    def _():
        _run(y_hbm, s * NCHUNK0, NG0)

    @pl.when(c == 1)
    def _():
        _run(y_hbm, NS * NCHUNK0 + s * NCHUNK1, NG1)

    plsc.subcore_barrier()
    sl = pl.ds(s * ROWS_PER_TILE, ROWS_PER_TILE)

    @pl.when(c == 0)
    def _():
        pltpu.sync_copy(acc_sh.at[sl], acc0_hbm.at[sl])

    @pl.when(c == 1)
    def _():
        pltpu.sync_copy(acc_sh.at[sl], acc1_hbm.at[sl])


BS = 2000


def _matmul_body(x_ref, w_ref, xw_ref):
    xw_ref[...] = jnp.dot(x_ref[...], w_ref[...],
                          preferred_element_type=jnp.float32)


def _matmul_tc(x_p, W_conv):
    return pl.pallas_call(
        _matmul_body,
        grid=(N // BS,),
        in_specs=[
            pl.BlockSpec((BS, D), lambda i: (i, 0)),
            pl.BlockSpec((D, D), lambda i: (0, 0)),
        ],
        out_specs=pl.BlockSpec((BS, D), lambda i: (i, 0)),
        out_shape=jax.ShapeDtypeStruct((N, D), jnp.float32),
    )(x_p, W_conv)


def _scale_body(xw_ref, d0_ref, d1_ref, y_ref):
    deg = d0_ref[...] + d1_ref[...] + 1.0
    dinv = lax.rsqrt(deg)
    y_ref[...] = xw_ref[...] * dinv


def _scale_tc(xw, deg0, deg1):
    grid = (N // BS,)
    return pl.pallas_call(
        _scale_body,
        grid=grid,
        in_specs=[
            pl.BlockSpec((BS, D), lambda i: (i, 0)),
            pl.BlockSpec((BS, 1), lambda i: (i, 0)),
            pl.BlockSpec((BS, 1), lambda i: (i, 0)),
        ],
        out_specs=pl.BlockSpec((BS, D), lambda i: (i, 0)),
        out_shape=jax.ShapeDtypeStruct((N, D), jnp.float32),
    )(xw, deg0, deg1)


def _combine_body(a0_ref, a1_ref, y_ref, d0_ref, d1_ref, bc_ref, wl_ref,
                  bl_ref, o_ref):
    deg = d0_ref[...] + d1_ref[...] + 1.0
    dinv = lax.rsqrt(deg)
    pre = (a0_ref[...] + a1_ref[...] + y_ref[...]) * dinv
    h = jnp.maximum(pre + bc_ref[...], 0.0)
    o_ref[...] = (
        jnp.dot(h, wl_ref[...], preferred_element_type=jnp.float32)
        + bl_ref[...]
    )


def _combine_tc(acc0, acc1, y, deg0, deg1, b_conv, W_lin, b_lin):
    grid = (N // BS,)
    blk = pl.BlockSpec((BS, D), lambda i: (i, 0))
    return pl.pallas_call(
        _combine_body,
        grid=grid,
        in_specs=[
            blk, blk, blk,
            pl.BlockSpec((BS, 1), lambda i: (i, 0)),
            pl.BlockSpec((BS, 1), lambda i: (i, 0)),
            pl.BlockSpec((1, D), lambda i: (0, 0)),
            pl.BlockSpec((D, D), lambda i: (0, 0)),
            pl.BlockSpec((1, D), lambda i: (0, 0)),
        ],
        out_specs=blk,
        out_shape=jax.ShapeDtypeStruct((N, D), jnp.float32),
    )(acc0, acc1, y, deg0, deg1, b_conv, W_lin, b_lin)


def kernel(x, edge_index, W_conv, b_conv, W_lin, b_lin):
    src = edge_index[0].astype(jnp.int32)
    dst = edge_index[1].astype(jnp.int32)
    dst_p = jnp.concatenate([dst, jnp.full((EPAD - E,), N, jnp.int32)])
    sd_3d = jnp.stack(
        [jnp.concatenate([src, jnp.zeros((EPAD2 - E,), jnp.int32)]
                         ).reshape(ECHUNKS, CH),
         jnp.concatenate([dst, jnp.full((EPAD2 - E,), N, jnp.int32)]
                         ).reshape(ECHUNKS, CH)],
        axis=1)

    xw = _matmul_tc(x, W_conv)
    deg0, deg1 = _deg_sc(dst_p)
    deg0 = deg0.reshape(NPAD, 1)
    deg1 = deg1.reshape(NPAD, 1)
    y = _scale_tc(xw, deg0, deg1)
    acc0, acc1 = _edge_sc(y, sd_3d)
    out = _combine_tc(acc0, acc1, y, deg0, deg1,
                      b_conv.reshape(1, D), W_lin, b_lin.reshape(1, D))
    return out


# speedup vs baseline: 1.2398x; 1.0014x over previous
import functools

import jax
import jax.numpy as jnp
from jax import lax
from jax.experimental import pallas as pl
from jax.experimental.pallas import tpu as pltpu
from jax.experimental.pallas import tpu_sc as plsc

N = 10000
D = 128
NC = 2
NS = 16
NW = NC * NS
NPAD = 10240
ROWS_PER_TILE = NPAD // NS
E = 320000
EPW = 10240
EPAD = NW * EPW
DCH = 128
DNCH = EPW // DCH
CH = 80
NCHUNK0 = 200
NCHUNK1 = 52
ECHUNKS = NS * (NCHUNK0 + NCHUNK1)
EPAD2 = ECHUNKS * CH
NB = 4

_mesh = plsc.VectorSubcoreMesh(
    core_axis_name="c", subcore_axis_name="s", num_cores=NC, num_subcores=NS
)


@functools.partial(
    pl.kernel,
    out_type=(
        jax.ShapeDtypeStruct((NPAD,), jnp.float32),
        jax.ShapeDtypeStruct((NPAD,), jnp.float32),
    ),
    mesh=_mesh,
    scratch_types=[
        pltpu.VMEM((NB, DCH), jnp.int32),
        pltpu.VMEM((DCH,), jnp.float32),
        pltpu.VMEM((DCH,), jnp.float32),
        pltpu.VMEM_SHARED((NPAD,), jnp.float32),
        [pltpu.SemaphoreType.DMA] * NB,
    ],
)
def _deg_sc(dst_hbm, deg0_hbm, deg1_hbm, idx_v, ones_v, zeros_v, deg_sh,
            dsems):
    c = lax.axis_index("c")
    s = lax.axis_index("s")
    wid = c * NS + s
    ebase = wid * EPW

    def idx_start(b, j):
        pltpu.async_copy(dst_hbm.at[pl.ds(ebase + j * DCH, DCH)],
                         idx_v.at[b], dsems[b])

    def idx_wait(b, j):
        pltpu.make_async_copy(dst_hbm.at[pl.ds(ebase + j * DCH, DCH)],
                              idx_v.at[b], dsems[b]).wait()

    for b in range(NB):
        idx_start(b, b)
    for k in range(DCH // 16):
        ones_v[pl.ds(k * 16, 16)] = jnp.ones((16,), jnp.float32)
        zeros_v[pl.ds(k * 16, 16)] = jnp.zeros((16,), jnp.float32)
    for k in range(ROWS_PER_TILE // DCH):
        pltpu.sync_copy(zeros_v, deg_sh.at[pl.ds(s * ROWS_PER_TILE + k * DCH, DCH)])
    plsc.subcore_barrier()

    def body(g, carry):
        for b in range(NB):
            j = g * NB + b
            idx_wait(b, j)
            pltpu.sync_copy(ones_v, deg_sh.at[idx_v.at[b]], add=True)
            idx_start(b, j + NB)
        return carry

    lax.fori_loop(0, DNCH // NB - 1, body, 0)
    for b in range(NB):
        j = DNCH - NB + b
        idx_wait(b, j)
        pltpu.sync_copy(ones_v, deg_sh.at[idx_v.at[b]], add=True)
    plsc.subcore_barrier()
    sl = pl.ds(s * ROWS_PER_TILE, ROWS_PER_TILE)

    @pl.when(c == 0)
    def _():
        pltpu.sync_copy(deg_sh.at[sl], deg0_hbm.at[sl])

    @pl.when(c == 1)
    def _():
        pltpu.sync_copy(deg_sh.at[sl], deg1_hbm.at[sl])


NG0 = NCHUNK0 // NB
NG1 = NCHUNK1 // NB


@functools.partial(
    pl.kernel,
    out_type=(
        jax.ShapeDtypeStruct((NPAD, D), jnp.float32),
        jax.ShapeDtypeStruct((NPAD, D), jnp.float32),
    ),
    mesh=_mesh,
    scratch_types=[
        pltpu.VMEM((NB, 2, CH), jnp.int32),
        pltpu.VMEM((NB, CH, D), jnp.float32),
        pltpu.VMEM_SHARED((NPAD, D), jnp.float32),
        [pltpu.SemaphoreType.DMA] * NB,
        [pltpu.SemaphoreType.DMA] * NB,
        [pltpu.SemaphoreType.DMA] * NB,
    ],
)
def _edge_sc(y_hbm, sd_hbm, acc0_hbm, acc1_hbm,
             sd_v, rows_v, acc_sh, isems, gsems, ssems):
    c = lax.axis_index("c")
    s = lax.axis_index("s")

    def _run(yref, cbase, ng):
        def idx_start(b, j):
            pltpu.async_copy(sd_hbm.at[cbase + j], sd_v.at[b], isems[b])

        def idx_wait(b, j):
            pltpu.make_async_copy(sd_hbm.at[cbase + j], sd_v.at[b],
                                  isems[b]).wait()

        def gather_start(b):
            pltpu.async_copy(yref.at[sd_v.at[b, 0]], rows_v.at[b], gsems[b])

        def gather_wait(b):
            pltpu.make_async_copy(yref.at[sd_v.at[b, 0]], rows_v.at[b],
                                  gsems[b]).wait()

        for b in range(NB):
            idx_start(b, b)
        for b in range(NB):
            idx_wait(b, b)
            gather_start(b)

        def scatter_start(b):
            return pltpu.async_copy(rows_v.at[b], acc_sh.at[sd_v.at[b, 1]],
                                    ssems[b], add=True)

        def body(g, carry):
            descs = []
            for b in range(NB):
                gather_wait(b)
                if b > 0:
                    descs[b - 1].wait()
                    idx_start(b - 1, (g + 1) * NB + b - 1)
                descs.append(scatter_start(b))
            descs[NB - 1].wait()
            idx_start(NB - 1, (g + 1) * NB + NB - 1)
            for b in range(NB):
                idx_wait(b, (g + 1) * NB + b)
                gather_start(b)
            return carry

        lax.fori_loop(0, ng - 1, body, 0)
        descs = []
        for b in range(NB):
            gather_wait(b)
            if b > 0:
                descs[b - 1].wait()
            descs.append(scatter_start(b))
        descs[NB - 1].wait()

    def zbody(r, carry):
        for k in range(D // 16):
            rows_v[0, r, pl.ds(k * 16, 16)] = jnp.zeros((16,), jnp.float32)
        return carry

    lax.fori_loop(0, CH, zbody, 0)
    for k in range(ROWS_PER_TILE // CH):
        pltpu.sync_copy(rows_v.at[0],
                        acc_sh.at[pl.ds(s * ROWS_PER_TILE + k * CH, CH)])
    plsc.subcore_barrier()

    @pl.when(c == 0)
    def _():
        _run(y_hbm, s * NCHUNK0, NG0)

    @pl.when(c == 1)
    def _():
        _run(y_hbm, NS * NCHUNK0 + s * NCHUNK1, NG1)

    plsc.subcore_barrier()
    sl = pl.ds(s * ROWS_PER_TILE, ROWS_PER_TILE)

    @pl.when(c == 0)
    def _():
        pltpu.sync_copy(acc_sh.at[sl], acc0_hbm.at[sl])

    @pl.when(c == 1)
    def _():
        pltpu.sync_copy(acc_sh.at[sl], acc1_hbm.at[sl])


BS = 5000


def _matmul_body(x_ref, w_ref, xw_ref):
    xw_ref[...] = jnp.dot(x_ref[...], w_ref[...],
                          preferred_element_type=jnp.float32)


def _matmul_tc(x_p, W_conv):
    return pl.pallas_call(
        _matmul_body,
        grid=(N // BS,),
        in_specs=[
            pl.BlockSpec((BS, D), lambda i: (i, 0)),
            pl.BlockSpec((D, D), lambda i: (0, 0)),
        ],
        out_specs=pl.BlockSpec((BS, D), lambda i: (i, 0)),
        out_shape=jax.ShapeDtypeStruct((N, D), jnp.float32),
    )(x_p, W_conv)


def _scale_body(xw_ref, d0_ref, d1_ref, y_ref):
    deg = d0_ref[...] + d1_ref[...] + 1.0
    dinv = lax.rsqrt(deg)
    y_ref[...] = xw_ref[...] * dinv


def _scale_tc(xw, deg0, deg1):
    grid = (N // BS,)
    return pl.pallas_call(
        _scale_body,
        grid=grid,
        in_specs=[
            pl.BlockSpec((BS, D), lambda i: (i, 0)),
            pl.BlockSpec((BS, 1), lambda i: (i, 0)),
            pl.BlockSpec((BS, 1), lambda i: (i, 0)),
        ],
        out_specs=pl.BlockSpec((BS, D), lambda i: (i, 0)),
        out_shape=jax.ShapeDtypeStruct((N, D), jnp.float32),
    )(xw, deg0, deg1)


def _combine_body(a0_ref, a1_ref, y_ref, d0_ref, d1_ref, bc_ref, wl_ref,
                  bl_ref, o_ref):
    deg = d0_ref[...] + d1_ref[...] + 1.0
    dinv = lax.rsqrt(deg)
    pre = (a0_ref[...] + a1_ref[...] + y_ref[...]) * dinv
    h = jnp.maximum(pre + bc_ref[...], 0.0)
    o_ref[...] = (
        jnp.dot(h, wl_ref[...], preferred_element_type=jnp.float32)
        + bl_ref[...]
    )


def _combine_tc(acc0, acc1, y, deg0, deg1, b_conv, W_lin, b_lin):
    grid = (N // BS,)
    blk = pl.BlockSpec((BS, D), lambda i: (i, 0))
    return pl.pallas_call(
        _combine_body,
        grid=grid,
        in_specs=[
            blk, blk, blk,
            pl.BlockSpec((BS, 1), lambda i: (i, 0)),
            pl.BlockSpec((BS, 1), lambda i: (i, 0)),
            pl.BlockSpec((1, D), lambda i: (0, 0)),
            pl.BlockSpec((D, D), lambda i: (0, 0)),
            pl.BlockSpec((1, D), lambda i: (0, 0)),
        ],
        out_specs=blk,
        out_shape=jax.ShapeDtypeStruct((N, D), jnp.float32),
    )(acc0, acc1, y, deg0, deg1, b_conv, W_lin, b_lin)


def kernel(x, edge_index, W_conv, b_conv, W_lin, b_lin):
    src = edge_index[0].astype(jnp.int32)
    dst = edge_index[1].astype(jnp.int32)
    dst_p = jnp.concatenate([dst, jnp.full((EPAD - E,), N, jnp.int32)])
    sd_3d = jnp.stack(
        [jnp.concatenate([src, jnp.zeros((EPAD2 - E,), jnp.int32)]
                         ).reshape(ECHUNKS, CH),
         jnp.concatenate([dst, jnp.full((EPAD2 - E,), N, jnp.int32)]
                         ).reshape(ECHUNKS, CH)],
        axis=1)

    xw = _matmul_tc(x, W_conv)
    deg0, deg1 = _deg_sc(dst_p)
    deg0 = deg0.reshape(NPAD, 1)
    deg1 = deg1.reshape(NPAD, 1)
    y = _scale_tc(xw, deg0, deg1)
    acc0, acc1 = _edge_sc(y, sd_3d)
    out = _combine_tc(acc0, acc1, y, deg0, deg1,
                      b_conv.reshape(1, D), W_lin, b_lin.reshape(1, D))
    return out
